# baseline (speedup 1.0000x reference)
import functools

import jax
import jax.numpy as jnp
from jax import lax
from jax.experimental import pallas as pl
from jax.experimental.pallas import tpu as pltpu
from jax.experimental.pallas import tpu_sc as plsc

D_MODEL = 64
_NC = 2
_NS = 16
_NW = _NC * _NS
_SUB = 128
_K = 8
_CHUNK = _K * _SUB


@functools.partial(jax.jit, static_argnames=("n_iter",))
def _gather(table, idx1d, n_iter):
    b = idx1d.shape[0]
    b_per_w = b // _NW
    mesh = plsc.VectorSubcoreMesh(core_axis_name="c", subcore_axis_name="s")

    @functools.partial(
        pl.kernel,
        mesh=mesh,
        compiler_params=pltpu.CompilerParams(use_tc_tiling_on_sc=False),
        out_type=jax.ShapeDtypeStruct((b, 2 * D_MODEL), jnp.float32),
        scratch_types=[
            pltpu.VMEM((_CHUNK,), jnp.int32),
            pltpu.VMEM((_CHUNK, D_MODEL), jnp.float32),
            pltpu.SemaphoreType.DMA,
        ],
    )
    def gather_kernel(table_hbm, idx_hbm, out_hbm, idx_v, rows_v, sem):
        wid = lax.axis_index("s") * _NC + lax.axis_index("c")
        base = wid * b_per_w

        def body(i, carry):
            off = pl.multiple_of(base + i * _CHUNK, _CHUNK)
            pltpu.sync_copy(idx_hbm.at[pl.ds(off, _CHUNK)], idx_v)
            copies = [
                pltpu.async_copy(
                    table_hbm.at[idx_v.at[pl.ds(j * _SUB, _SUB)]],
                    rows_v.at[pl.ds(j * _SUB, _SUB)],
                    sem,
                )
                for j in range(_K)
            ]
            for c in copies:
                c.wait()
            pltpu.sync_copy(
                rows_v,
                out_hbm.at[pl.ds(off, _CHUNK), pl.ds(0, D_MODEL)],
            )
            return carry

        lax.fori_loop(0, n_iter, body, 0)

    return gather_kernel(table, idx1d)


def kernel(x, table):
    s0, s1 = x.shape
    b = s0 * s1
    idx1d = jnp.clip(x.reshape(-1).astype(jnp.int32), 0, table.shape[0] - 1)
    n_iter = b // (_NW * _CHUNK)
    out128 = _gather(table, idx1d, n_iter)
    return out128.reshape(s0, s1, 2 * D_MODEL)[:, :, :D_MODEL]


# speedup vs baseline: 1.0135x; 1.0135x over previous
import functools

import jax
import jax.numpy as jnp
from jax import lax
from jax.experimental import pallas as pl
from jax.experimental.pallas import tpu as pltpu
from jax.experimental.pallas import tpu_sc as plsc

D_MODEL = 64
_NC = 2
_NS = 16
_NW = _NC * _NS
_SUB = 128
_K = 5
_CHUNK = _K * _SUB


@functools.partial(jax.jit, static_argnames=("n_iter",))
def _gather(table, idx1d, n_iter):
    b = idx1d.shape[0]
    b_per_w = b // _NW
    mesh = plsc.VectorSubcoreMesh(core_axis_name="c", subcore_axis_name="s")

    @functools.partial(
        pl.kernel,
        mesh=mesh,
        compiler_params=pltpu.CompilerParams(use_tc_tiling_on_sc=False),
        out_type=jax.ShapeDtypeStruct((b, 2 * D_MODEL), jnp.float32),
        scratch_types=[
            pltpu.VMEM((2, _CHUNK), jnp.int32),
            pltpu.VMEM((2, _CHUNK, D_MODEL), jnp.float32),
            pltpu.SemaphoreType.DMA,
            pltpu.SemaphoreType.DMA,
        ],
    )
    def gather_kernel(table_hbm, idx_hbm, out_hbm, idx_v, rows_v, sg0, sg1):
        sems = (sg0, sg1)
        wid = lax.axis_index("s") * _NC + lax.axis_index("c")
        base = wid * b_per_w
        last = b - _CHUNK

        def stage(i, p):
            off = pl.multiple_of(lax.min(base + i * _CHUNK, last), _CHUNK)
            pltpu.sync_copy(idx_hbm.at[pl.ds(off, _CHUNK)], idx_v.at[p])

        def fire(p):
            for j in range(_K):
                pltpu.async_copy(
                    table_hbm.at[idx_v.at[p, pl.ds(j * _SUB, _SUB)]],
                    rows_v.at[p, pl.ds(j * _SUB, _SUB)],
                    sems[p],
                )

        def drain(p):
            pltpu.make_async_copy(
                table_hbm.at[pl.ds(0, _CHUNK)], rows_v.at[p], sems[p]
            ).wait()

        stage(0, 0)
        fire(0)

        def body(t, carry):
            for p in (0, 1):
                i = 2 * t + p
                drain(p)
                stage(i + 1, p ^ 1)
                fire(p ^ 1)
                off = pl.multiple_of(base + i * _CHUNK, _CHUNK)
                pltpu.sync_copy(
                    rows_v.at[p],
                    out_hbm.at[pl.ds(off, _CHUNK), pl.ds(0, D_MODEL)],
                )
            return carry

        lax.fori_loop(0, n_iter // 2, body, 0)
        drain(0)

    return gather_kernel(table, idx1d)


def kernel(x, table):
    s0, s1 = x.shape
    b = s0 * s1
    idx1d = x.reshape(-1).astype(jnp.int32)
    n_iter = b // (_NW * _CHUNK)
    out128 = _gather(table, idx1d, n_iter)
    return out128.reshape(s0, s1, 2 * D_MODEL)[:, :, :D_MODEL]


# speedup vs baseline: 1.0430x; 1.0291x over previous
import functools

import jax
import jax.numpy as jnp
from jax import lax
from jax.experimental import pallas as pl
from jax.experimental.pallas import tpu as pltpu
from jax.experimental.pallas import tpu_sc as plsc

D_MODEL = 64
_NC = 2
_NS = 16
_NW = _NC * _NS
_SUB = 128
_K = 5
_CHUNK = _K * _SUB


@functools.partial(jax.jit, static_argnames=("n_iter",))
def _gather(table, idx1d, n_iter):
    b = idx1d.shape[0]
    b_per_w = b // _NW
    mesh = plsc.VectorSubcoreMesh(core_axis_name="c", subcore_axis_name="s")

    @functools.partial(
        pl.kernel,
        mesh=mesh,
        compiler_params=pltpu.CompilerParams(use_tc_tiling_on_sc=False),
        out_type=jax.ShapeDtypeStruct((b, 2 * D_MODEL), jnp.float32),
        scratch_types=[
            pltpu.VMEM((2, _CHUNK), jnp.int32),
            pltpu.VMEM((2, _CHUNK, D_MODEL), jnp.float32),
            pltpu.SemaphoreType.DMA,
            pltpu.SemaphoreType.DMA,
            pltpu.SemaphoreType.DMA,
            pltpu.SemaphoreType.DMA,
        ],
    )
    def gather_kernel(table_hbm, idx_hbm, out_hbm, idx_v, rows_v,
                      sg0, sg1, si0, si1):
        sems = (sg0, sg1)
        isems = (si0, si1)
        wid = lax.axis_index("s") * _NC + lax.axis_index("c")
        base = wid * b_per_w
        last = b - _CHUNK

        def stage(i, p):
            off = pl.multiple_of(lax.min(base + i * _CHUNK, last), _CHUNK)
            pltpu.async_copy(idx_hbm.at[pl.ds(off, _CHUNK)], idx_v.at[p],
                             isems[p])

        def stage_wait(p):
            pltpu.make_async_copy(
                idx_hbm.at[pl.ds(0, _CHUNK)], idx_v.at[p], isems[p]
            ).wait()

        def fire(p):
            for j in range(_K):
                pltpu.async_copy(
                    table_hbm.at[idx_v.at[p, pl.ds(j * _SUB, _SUB)]],
                    rows_v.at[p, pl.ds(j * _SUB, _SUB)],
                    sems[p],
                )

        def drain(p):
            pltpu.make_async_copy(
                table_hbm.at[pl.ds(0, _CHUNK)], rows_v.at[p], sems[p]
            ).wait()

        stage(0, 0)
        stage_wait(0)
        fire(0)
        stage(1, 1)

        def body(t, carry):
            for p in (0, 1):
                i = 2 * t + p
                drain(p)
                stage_wait(p ^ 1)
                fire(p ^ 1)
                stage(i + 2, p)
                off = pl.multiple_of(base + i * _CHUNK, _CHUNK)
                pltpu.sync_copy(
                    rows_v.at[p],
                    out_hbm.at[pl.ds(off, _CHUNK), pl.ds(0, D_MODEL)],
                )
            return carry

        lax.fori_loop(0, n_iter // 2, body, 0)
        drain(0)
        stage_wait(1)

    return gather_kernel(table, idx1d)


def kernel(x, table):
    s0, s1 = x.shape
    b = s0 * s1
    idx1d = x.reshape(-1).astype(jnp.int32)
    n_iter = b // (_NW * _CHUNK)
    out128 = _gather(table, idx1d, n_iter)
    return out128.reshape(s0, s1, 2 * D_MODEL)[:, :, :D_MODEL]
